# baseline (speedup 1.0000x reference)
import functools

import jax
import jax.numpy as jnp
from jax import lax
from jax.experimental import pallas as pl
from jax.experimental.pallas import tpu as pltpu
from jax.experimental.pallas import tpu_sc as plsc

NUM_EMB = 100000
DIM = 128
DIM_W = DIM // 2
BATCH = 4096
HIST = 200
TOTAL = BATCH * HIST

NC, NS = 2, 16
NW = NC * NS
PER_W = TOTAL // NW
CHUNK = 128
N_CHUNKS = PER_W // CHUNK


def _cast_body(w_ref, o_ref):
    o_ref[...] = w_ref[...].astype(jnp.bfloat16)


def _cast_table(weight):
    rows_blk = 2000
    return pl.pallas_call(
        _cast_body,
        out_shape=jax.ShapeDtypeStruct((NUM_EMB, DIM), jnp.bfloat16),
        grid=(NUM_EMB // rows_blk,),
        in_specs=[pl.BlockSpec((rows_blk, DIM), lambda i: (i, 0))],
        out_specs=pl.BlockSpec((rows_blk, DIM), lambda i: (i, 0)),
    )(weight)


_mesh = plsc.VectorSubcoreMesh(core_axis_name="c", subcore_axis_name="s")


@functools.partial(
    pl.kernel,
    out_type=jax.ShapeDtypeStruct((TOTAL, DIM_W), jnp.int32),
    mesh=_mesh,
    scratch_types=[
        pltpu.VMEM((PER_W,), jnp.int32),
        pltpu.VMEM((CHUNK, DIM_W), jnp.int32),
        pltpu.SemaphoreType.DMA,
    ],
    compiler_params=pltpu.CompilerParams(use_tc_tiling_on_sc=False),
)
def _sc_gather(table_hbm, idx_hbm, out_hbm, idx_v, rows_v, sem):
    wid = lax.axis_index("s") * NC + lax.axis_index("c")
    base = wid * PER_W
    pltpu.sync_copy(idx_hbm.at[pl.ds(base, PER_W)], idx_v)

    def body(j, carry):
        off = j * CHUNK
        pltpu.async_copy(
            table_hbm.at[idx_v.at[pl.ds(off, CHUNK)]], rows_v, sem
        ).wait()
        pltpu.sync_copy(rows_v, out_hbm.at[pl.ds(base + off, CHUNK)])
        return carry

    lax.fori_loop(0, N_CHUNKS, body, 0)


def kernel(x, weight):
    wb = _cast_table(weight)
    table_i32 = lax.bitcast_convert_type(
        wb.reshape(NUM_EMB, DIM_W, 2), jnp.int32
    )
    idx_flat = x.reshape(TOTAL)
    out_i32 = _sc_gather(table_i32, idx_flat)
    out = lax.bitcast_convert_type(out_i32, jnp.bfloat16)
    return out.reshape(BATCH, HIST, DIM)


# speedup vs baseline: 9.2293x; 9.2293x over previous
import functools

import jax
import jax.numpy as jnp
from jax import lax
from jax.experimental import pallas as pl
from jax.experimental.pallas import tpu as pltpu
from jax.experimental.pallas import tpu_sc as plsc

NUM_EMB = 100000
DIM = 128
BATCH = 4096
HIST = 200
TOTAL = BATCH * HIST

NC, NS = 2, 16
NW = NC * NS
PER_W = TOTAL // NW
CHUNK = 128
NBUF = 4
N_STEPS = PER_W // (CHUNK * NBUF)


def _cast_body(v_ref, o_ref):
    o_ref[...] = v_ref[...].astype(jnp.bfloat16)


def _cast_rows(rows_f32):
    rows_blk = 8192
    return pl.pallas_call(
        _cast_body,
        out_shape=jax.ShapeDtypeStruct((TOTAL, DIM), jnp.bfloat16),
        grid=(TOTAL // rows_blk,),
        in_specs=[pl.BlockSpec((rows_blk, DIM), lambda i: (i, 0))],
        out_specs=pl.BlockSpec((rows_blk, DIM), lambda i: (i, 0)),
    )(rows_f32)


_mesh = plsc.VectorSubcoreMesh(core_axis_name="c", subcore_axis_name="s")


@functools.partial(
    pl.kernel,
    out_type=jax.ShapeDtypeStruct((TOTAL, DIM), jnp.float32),
    mesh=_mesh,
    scratch_types=[
        pltpu.VMEM((PER_W,), jnp.int32),
        pltpu.VMEM((NBUF, CHUNK, DIM), jnp.float32),
        pltpu.SemaphoreType.DMA,
        pltpu.SemaphoreType.DMA,
    ],
)
def _sc_gather(table_hbm, idx_hbm, out_hbm, idx_v, rows_v, gsem, ssem):
    wid = lax.axis_index("s") * NC + lax.axis_index("c")
    base = wid * PER_W
    pltpu.sync_copy(idx_hbm.at[pl.ds(base, PER_W)], idx_v)

    def body(i, carry):
        step = i * (CHUNK * NBUF)
        gathers = []
        for b in range(NBUF):
            off = step + b * CHUNK
            gathers.append(
                pltpu.async_copy(
                    table_hbm.at[idx_v.at[pl.ds(off, CHUNK)]],
                    rows_v.at[b],
                    gsem,
                )
            )
        stores = []
        for b in range(NBUF):
            off = step + b * CHUNK
            gathers[b].wait()
            stores.append(
                pltpu.async_copy(
                    rows_v.at[b],
                    out_hbm.at[pl.ds(base + off, CHUNK)],
                    ssem,
                )
            )
        for st in stores:
            st.wait()
        return carry

    lax.fori_loop(0, N_STEPS, body, 0)


def kernel(x, weight):
    idx_flat = x.reshape(TOTAL)
    rows_f32 = _sc_gather(weight, idx_flat)
    out = _cast_rows(rows_f32)
    return out.reshape(BATCH, HIST, DIM)
